# Initial kernel scaffold; baseline (speedup 1.0000x reference)
import jax
import jax.numpy as jnp
from jax.experimental import pallas as pl


def kernel(features, edge_index, edge_weight, W1, b1, W2, b2, W3, b3, W4, b4, W5, b5):
    raise NotImplementedError("write your pallas kernel here")


# speedup vs baseline: 3.0577x; 3.0577x over previous
import jax
import jax.numpy as jnp
from jax import lax
from jax.experimental import pallas as pl
from jax.experimental.pallas import tpu as pltpu
from jax.experimental.pallas import tpu_sc as plsc

_N = 10000
_E = 320000
_F = 128
_NC = 2
_NS = 16
_NW = _NC * _NS
_CHUNK = 128
_NCH = 80
_EPW = _CHUNK * _NCH
_EPAD = _NW * _EPW
_NPAD = 10240
_RPT = _NPAD // _NS
_GTR = lax.GatherDimensionNumbers(offset_dims=(), collapsed_slice_dims=(0,),
                                  start_index_map=(0,))


def _spmm_body(u_hbm, src_hbm, dst_hbm, w_hbm, out_hbm,
               src_v, dst_v, w_v, gbuf, acc, gsem):
    c = lax.axis_index("c")
    s = lax.axis_index("s")
    wid = c * _NS + s

    zero16 = jnp.zeros((16,), jnp.float32)
    for r in range(_CHUNK):
        for f in range(_F // 16):
            gbuf[r, pl.ds(f * 16, 16)] = zero16
    base = s * _RPT
    for k in range(_RPT // _CHUNK):
        pltpu.sync_copy(gbuf, acc.at[pl.ds(base + k * _CHUNK, _CHUNK)])

    pltpu.sync_copy(src_hbm.at[wid], src_v)
    pltpu.sync_copy(dst_hbm.at[wid], dst_v)
    pltpu.sync_copy(w_hbm.at[wid], w_v)
    plsc.subcore_barrier()

    def chunk_body(j, carry):
        pltpu.async_copy(u_hbm.at[src_v.at[j]], gbuf, gsem).wait()
        for g in range(_CHUNK // 16):
            w16 = w_v[pl.ds(j * _CHUNK + g * 16, 16)]
            for e in range(16):
                wb = lax.gather(
                    w16, jnp.full((16, 1), e, jnp.int32),
                    _GTR, slice_sizes=(1,),
                    mode=lax.GatherScatterMode.PROMISE_IN_BOUNDS)
                for f in range(_F // 16):
                    sl = pl.ds(f * 16, 16)
                    row = g * 16 + e
                    gbuf[row, sl] = gbuf[row, sl] * wb
        pltpu.sync_copy(gbuf, acc.at[dst_v.at[j]], add=True)
        return carry

    lax.fori_loop(0, _NCH, chunk_body, 0)
    plsc.subcore_barrier()

    pltpu.sync_copy(acc.at[pl.ds(base, _RPT)],
                    out_hbm.at[c, pl.ds(base, _RPT)])


_spmm = pl.kernel(
    _spmm_body,
    out_type=jax.ShapeDtypeStruct((_NC, _NPAD, _F), jnp.float32),
    mesh=plsc.VectorSubcoreMesh(core_axis_name="c", subcore_axis_name="s"),
    scratch_types=[
        pltpu.VMEM((_NCH, _CHUNK), jnp.int32),
        pltpu.VMEM((_NCH, _CHUNK), jnp.int32),
        pltpu.VMEM((_EPW,), jnp.float32),
        pltpu.VMEM((_CHUNK, _F), jnp.float32),
        pltpu.VMEM_SHARED((_NPAD, _F), jnp.float32),
        pltpu.SemaphoreType.DMA,
    ],
)

_BLK = 1000


def _mm_body(x_ref, w_ref, o_ref):
    o_ref[...] = jnp.dot(x_ref[...], w_ref[...],
                         preferred_element_type=jnp.float32)


def _fuse_mm_body(p_ref, b_ref, w_ref, o_ref):
    h = jnp.maximum(p_ref[0] + p_ref[1] + b_ref[...], 0.0)
    o_ref[...] = jnp.dot(h, w_ref[...], preferred_element_type=jnp.float32)


def _relu_body(p_ref, b_ref, o_ref):
    o_ref[...] = jnp.maximum(p_ref[0] + p_ref[1] + b_ref[...], 0.0)


def _mm_bias_body(p_ref, w_ref, b_ref, o_ref):
    o_ref[...] = jnp.dot(p_ref[0] + p_ref[1], w_ref[...],
                         preferred_element_type=jnp.float32) + b_ref[...]


def _lsm_body(x_ref, o_ref):
    x = x_ref[...]
    m = jnp.max(x, axis=0, keepdims=True)
    lse = m + jnp.log(jnp.sum(jnp.exp(x - m), axis=0, keepdims=True))
    o_ref[...] = x - lse


def _tc_mm(x, w):
    return pl.pallas_call(
        _mm_body,
        grid=(_N // _BLK,),
        in_specs=[pl.BlockSpec((_BLK, _F), lambda i: (i, 0)),
                  pl.BlockSpec((_F, _F), lambda i: (0, 0))],
        out_specs=pl.BlockSpec((_BLK, _F), lambda i: (i, 0)),
        out_shape=jax.ShapeDtypeStruct((_N, _F), jnp.float32),
    )(x, w)


def _tc_fuse_mm(p, b, w):
    return pl.pallas_call(
        _fuse_mm_body,
        grid=(_N // _BLK,),
        in_specs=[pl.BlockSpec((_NC, _BLK, _F), lambda i: (0, i, 0)),
                  pl.BlockSpec((_F,), lambda i: (0,)),
                  pl.BlockSpec((_F, _F), lambda i: (0, 0))],
        out_specs=pl.BlockSpec((_BLK, _F), lambda i: (i, 0)),
        out_shape=jax.ShapeDtypeStruct((_N, _F), jnp.float32),
    )(p, b, w)


def _tc_relu(p, b):
    return pl.pallas_call(
        _relu_body,
        grid=(_N // _BLK,),
        in_specs=[pl.BlockSpec((_NC, _BLK, _F), lambda i: (0, i, 0)),
                  pl.BlockSpec((_F,), lambda i: (0,))],
        out_specs=pl.BlockSpec((_BLK, _F), lambda i: (i, 0)),
        out_shape=jax.ShapeDtypeStruct((_N, _F), jnp.float32),
    )(p, b)


def _tc_mm_bias(p, w, b):
    nout = w.shape[1]
    return pl.pallas_call(
        _mm_bias_body,
        grid=(_N // _BLK,),
        in_specs=[pl.BlockSpec((_NC, _BLK, _F), lambda i: (0, i, 0)),
                  pl.BlockSpec((_F, nout), lambda i: (0, 0)),
                  pl.BlockSpec((nout,), lambda i: (0,))],
        out_specs=pl.BlockSpec((_BLK, nout), lambda i: (i, 0)),
        out_shape=jax.ShapeDtypeStruct((_N, nout), jnp.float32),
    )(p, w, b)


def _tc_lsm(x):
    nout = x.shape[1]
    return pl.pallas_call(
        _lsm_body,
        in_specs=[pl.BlockSpec((_N, nout), lambda: (0, 0))],
        out_specs=pl.BlockSpec((_N, nout), lambda: (0, 0)),
        out_shape=jax.ShapeDtypeStruct((_N, nout), jnp.float32),
    )(x)


def kernel(features, edge_index, edge_weight, W1, b1, W2, b2, W3, b3,
           W4, b4, W5, b5):
    pad = _EPAD - _E
    src = jnp.concatenate([edge_index[0], jnp.zeros((pad,), jnp.int32)])
    dst = jnp.concatenate([edge_index[1], jnp.zeros((pad,), jnp.int32)])
    w = jnp.concatenate([edge_weight, jnp.zeros((pad,), jnp.float32)])
    src3 = src.reshape(_NW, _NCH, _CHUNK)
    dst3 = dst.reshape(_NW, _NCH, _CHUNK)
    w3 = w.reshape(_NW, _EPW)

    u = _tc_mm(features, W1)
    for (b, W) in ((b1, W2), (b2, W3), (b3, W4)):
        p = _spmm(u, src3, dst3, w3)
        u = _tc_fuse_mm(p, b, W)
    p = _spmm(u, src3, dst3, w3)
    h4 = _tc_relu(p, b4)
    q = _spmm(h4, src3, dst3, w3)
    logits = _tc_mm_bias(q, W5, b5)
    return _tc_lsm(logits)


# speedup vs baseline: 3.1505x; 1.0304x over previous
import jax
import jax.numpy as jnp
from jax import lax
from jax.experimental import pallas as pl
from jax.experimental.pallas import tpu as pltpu
from jax.experimental.pallas import tpu_sc as plsc

_N = 10000
_E = 320000
_F = 128
_NC = 2
_NS = 16
_NW = _NC * _NS
_CHUNK = 128
_NCH = 80
_EPW = _CHUNK * _NCH
_EPAD = _NW * _EPW
_NPAD = 10240
_RPT = _NPAD // _NS
_GTR = lax.GatherDimensionNumbers(offset_dims=(), collapsed_slice_dims=(0,),
                                  start_index_map=(0,))
_BLKCH = 16


def _spmm_body(u_hbm, src_hbm, dst_hbm, w_hbm, out_hbm,
               src_v, dst_b, w_b, gbuf, acc, gsem):
    c = lax.axis_index("c")
    s = lax.axis_index("s")
    wid = c * _NS + s

    zero16 = jnp.zeros((16,), jnp.float32)

    def zrow(r, carry):
        for f in range(_F // 16):
            gbuf[0, r, pl.ds(f * 16, 16)] = zero16
        return carry

    lax.fori_loop(0, _CHUNK, zrow, 0)
    base = s * _RPT
    for k in range(_RPT // _CHUNK):
        pltpu.sync_copy(gbuf.at[0], acc.at[pl.ds(base + k * _CHUNK, _CHUNK)])

    pltpu.sync_copy(src_hbm.at[wid], src_v)
    plsc.subcore_barrier()

    pltpu.async_copy(u_hbm.at[src_v.at[0]], gbuf.at[0], gsem.at[0])
    pltpu.async_copy(u_hbm.at[src_v.at[1]], gbuf.at[1], gsem.at[1])

    def chunk_body(j, carry):
        b = lax.rem(j, 2)
        jm = lax.rem(j, _BLKCH)

        @pl.when(jm == 0)
        def _():
            blk = lax.div(j, _BLKCH)
            pltpu.sync_copy(dst_hbm.at[wid, pl.ds(blk * _BLKCH, _BLKCH)],
                            dst_b)
            pltpu.sync_copy(
                w_hbm.at[wid, pl.ds(blk * _BLKCH * _CHUNK,
                                    _BLKCH * _CHUNK)], w_b)

        pltpu.make_async_copy(u_hbm.at[src_v.at[j]], gbuf.at[b],
                              gsem.at[b]).wait()

        def grp(g, carry2):
            w16 = w_b[pl.ds(jm * _CHUNK + g * 16, 16)]
            for e in range(16):
                wb = lax.gather(
                    w16, jnp.full((16, 1), e, jnp.int32),
                    _GTR, slice_sizes=(1,),
                    mode=lax.GatherScatterMode.PROMISE_IN_BOUNDS)
                row = g * 16 + e
                for f in range(_F // 16):
                    sl = pl.ds(f * 16, 16)
                    gbuf[b, row, sl] = gbuf[b, row, sl] * wb
            return carry2

        lax.fori_loop(0, _CHUNK // 16, grp, 0)
        pltpu.sync_copy(gbuf.at[b], acc.at[dst_b.at[jm]], add=True)

        @pl.when(j < _NCH - 2)
        def _():
            pltpu.async_copy(u_hbm.at[src_v.at[j + 2]], gbuf.at[b],
                             gsem.at[b])

        return carry

    lax.fori_loop(0, _NCH, chunk_body, 0)
    plsc.subcore_barrier()

    pltpu.sync_copy(acc.at[pl.ds(base, _RPT)],
                    out_hbm.at[c, pl.ds(base, _RPT)])


_spmm = pl.kernel(
    _spmm_body,
    out_type=jax.ShapeDtypeStruct((_NC, _NPAD, _F), jnp.float32),
    mesh=plsc.VectorSubcoreMesh(core_axis_name="c", subcore_axis_name="s"),
    scratch_types=[
        pltpu.VMEM((_NCH, _CHUNK), jnp.int32),
        pltpu.VMEM((_BLKCH, _CHUNK), jnp.int32),
        pltpu.VMEM((_BLKCH * _CHUNK,), jnp.float32),
        pltpu.VMEM((2, _CHUNK, _F), jnp.float32),
        pltpu.VMEM_SHARED((_NPAD, _F), jnp.float32),
        pltpu.SemaphoreType.DMA((2,)),
    ],
)

_BLK = 1000


def _mm_body(x_ref, w_ref, o_ref):
    o_ref[...] = jnp.dot(x_ref[...], w_ref[...],
                         preferred_element_type=jnp.float32)


def _fuse_mm_body(p_ref, b_ref, w_ref, o_ref):
    h = jnp.maximum(p_ref[0] + p_ref[1] + b_ref[...], 0.0)
    o_ref[...] = jnp.dot(h, w_ref[...], preferred_element_type=jnp.float32)


def _relu_body(p_ref, b_ref, o_ref):
    o_ref[...] = jnp.maximum(p_ref[0] + p_ref[1] + b_ref[...], 0.0)


def _mm_bias_body(p_ref, w_ref, b_ref, o_ref):
    o_ref[...] = jnp.dot(p_ref[0] + p_ref[1], w_ref[...],
                         preferred_element_type=jnp.float32) + b_ref[...]


def _lsm_body(x_ref, o_ref):
    x = x_ref[...]
    m = jnp.max(x, axis=0, keepdims=True)
    lse = m + jnp.log(jnp.sum(jnp.exp(x - m), axis=0, keepdims=True))
    o_ref[...] = x - lse


def _tc_mm(x, w):
    return pl.pallas_call(
        _mm_body,
        grid=(_N // _BLK,),
        in_specs=[pl.BlockSpec((_BLK, _F), lambda i: (i, 0)),
                  pl.BlockSpec((_F, _F), lambda i: (0, 0))],
        out_specs=pl.BlockSpec((_BLK, _F), lambda i: (i, 0)),
        out_shape=jax.ShapeDtypeStruct((_N, _F), jnp.float32),
    )(x, w)


def _tc_fuse_mm(p, b, w):
    return pl.pallas_call(
        _fuse_mm_body,
        grid=(_N // _BLK,),
        in_specs=[pl.BlockSpec((_NC, _BLK, _F), lambda i: (0, i, 0)),
                  pl.BlockSpec((_F,), lambda i: (0,)),
                  pl.BlockSpec((_F, _F), lambda i: (0, 0))],
        out_specs=pl.BlockSpec((_BLK, _F), lambda i: (i, 0)),
        out_shape=jax.ShapeDtypeStruct((_N, _F), jnp.float32),
    )(p, b, w)


def _tc_relu(p, b):
    return pl.pallas_call(
        _relu_body,
        grid=(_N // _BLK,),
        in_specs=[pl.BlockSpec((_NC, _BLK, _F), lambda i: (0, i, 0)),
                  pl.BlockSpec((_F,), lambda i: (0,))],
        out_specs=pl.BlockSpec((_BLK, _F), lambda i: (i, 0)),
        out_shape=jax.ShapeDtypeStruct((_N, _F), jnp.float32),
    )(p, b)


def _tc_mm_bias(p, w, b):
    nout = w.shape[1]
    return pl.pallas_call(
        _mm_bias_body,
        grid=(_N // _BLK,),
        in_specs=[pl.BlockSpec((_NC, _BLK, _F), lambda i: (0, i, 0)),
                  pl.BlockSpec((_F, nout), lambda i: (0, 0)),
                  pl.BlockSpec((nout,), lambda i: (0,))],
        out_specs=pl.BlockSpec((_BLK, nout), lambda i: (i, 0)),
        out_shape=jax.ShapeDtypeStruct((_N, nout), jnp.float32),
    )(p, w, b)


def _tc_lsm(x):
    nout = x.shape[1]
    return pl.pallas_call(
        _lsm_body,
        in_specs=[pl.BlockSpec((_N, nout), lambda: (0, 0))],
        out_specs=pl.BlockSpec((_N, nout), lambda: (0, 0)),
        out_shape=jax.ShapeDtypeStruct((_N, nout), jnp.float32),
    )(x)


def kernel(features, edge_index, edge_weight, W1, b1, W2, b2, W3, b3,
           W4, b4, W5, b5):
    pad = _EPAD - _E
    src = jnp.concatenate([edge_index[0], jnp.zeros((pad,), jnp.int32)])
    dst = jnp.concatenate([edge_index[1], jnp.zeros((pad,), jnp.int32)])
    w = jnp.concatenate([edge_weight, jnp.zeros((pad,), jnp.float32)])
    src3 = src.reshape(_NW, _NCH, _CHUNK)
    dst3 = dst.reshape(_NW, _NCH, _CHUNK)
    w3 = w.reshape(_NW, _EPW)

    u = _tc_mm(features, W1)
    for (b, W) in ((b1, W2), (b2, W3), (b3, W4)):
        p = _spmm(u, src3, dst3, w3)
        u = _tc_fuse_mm(p, b, W)
    p = _spmm(u, src3, dst3, w3)
    h4 = _tc_relu(p, b4)
    q = _spmm(h4, src3, dst3, w3)
    logits = _tc_mm_bias(q, W5, b5)
    return _tc_lsm(logits)
